# baseline (speedup 1.0000x reference)
import functools

import jax
import jax.numpy as jnp
from jax import lax
from jax.experimental import pallas as pl
from jax.experimental.pallas import tpu as pltpu
from jax.experimental.pallas import tpu_sc as plsc

C = 128
H = 256
F = 256
ITERS = 2
ET = 4
NSEM = 57
IB = 16

SC_NC, SC_NS, SC_L = 2, 16, 16
NW = SC_NC * SC_NS
CCOLS = C * H // NW
SCCHUNK = 128
NCHUNKS = CCOLS // SCCHUNK
NGRP = SCCHUNK // SC_L


def _sc_matvec_body(parent_hbm, wp_hbm, bp_hbm, child_hbm,
                    pbuf, bbuf, wb0, wb1, obuf, sem0, sem1):
    wid = lax.axis_index("s") * SC_NC + lax.axis_index("c")
    base = wid * CCOLS
    pltpu.sync_copy(parent_hbm, pbuf)
    pltpu.sync_copy(bp_hbm.at[pl.ds(base, CCOLS)], bbuf)
    bufs = (wb0, wb1)
    sems = (sem0, sem1)

    def chunk_copy(ch, buf, sem):
        return pltpu.make_async_copy(
            wp_hbm.at[:, pl.ds(base + ch * SCCHUNK, SCCHUNK)], buf, sem)

    chunk_copy(0, wb0, sem0).start()
    for ch in range(NCHUNKS):
        buf, sem = bufs[ch % 2], sems[ch % 2]
        chunk_copy(ch, buf, sem).wait()
        if ch + 1 < NCHUNKS:
            chunk_copy(ch + 1, bufs[(ch + 1) % 2], sems[(ch + 1) % 2]).start()

        def qbody(q, accs):
            pv = pbuf[pl.ds(q * SC_L, SC_L)]
            for d in range(SC_L):
                p = pv[d]
                f = q * SC_L + d
                accs = tuple(accs[g] + p * buf[f, pl.ds(g * SC_L, SC_L)]
                             for g in range(NGRP))
            return accs

        accs = lax.fori_loop(
            0, F // SC_L, qbody,
            tuple(bbuf[pl.ds(ch * SCCHUNK + g * SC_L, SC_L)]
                  for g in range(NGRP)))
        for g in range(NGRP):
            obuf[pl.ds(ch * SCCHUNK + g * SC_L, SC_L)] = \
                jnp.maximum(accs[g], 0.0)
    pltpu.sync_copy(obuf, child_hbm.at[pl.ds(base, CCOLS)])


_sc_matvec = functools.partial(
    pl.kernel,
    out_type=jax.ShapeDtypeStruct((C * H,), jnp.float32),
    mesh=plsc.VectorSubcoreMesh(core_axis_name="c", subcore_axis_name="s",
                                num_cores=SC_NC, num_subcores=SC_NS),
    scratch_types=[
        pltpu.VMEM((F,), jnp.float32),
        pltpu.VMEM((CCOLS,), jnp.float32),
        pltpu.VMEM((F, SCCHUNK), jnp.float32),
        pltpu.VMEM((F, SCCHUNK), jnp.float32),
        pltpu.VMEM((CCOLS,), jnp.float32),
        pltpu.SemaphoreType.DMA,
        pltpu.SemaphoreType.DMA,
    ])(_sc_matvec_body)


def _body(child_ref, wex_ref, bex_ref, wel_ref, bel_ref,
          wee_ref, bee_ref, wne_ref, bne_ref, wch_ref, bch_ref,
          wsem_ref, bsem_ref, wch2_ref, bch2_ref,
          out_ref, sem_ref, exists_ref, elog_ref):
    if True:
        child = child_ref[...]

        exl = jnp.dot(child, wex_ref[...],
                      preferred_element_type=jnp.float32) + bex_ref[...]
        exists_ref[...] = exl[None]
        exf = (exl[:, 0] > 0.0).astype(jnp.float32)

        wel = wel_ref[...]
        ea = jnp.dot(child, wel[:H],
                     preferred_element_type=jnp.float32) + bel_ref[...]
        eb = jnp.dot(child, wel[H:], preferred_element_type=jnp.float32)
        cnt_rows = []
        for ib in range(C // IB):
            el = jnp.maximum(ea[ib * IB:(ib + 1) * IB][:, None, :]
                             + eb[None, :, :], 0.0)
            lb = jnp.dot(el.reshape(IB * C, H), wee_ref[...],
                         preferred_element_type=jnp.float32) + bee_ref[...]
            lb3 = lb.reshape(IB, C, ET)
            elog_ref[0, pl.ds(ib * IB, IB), :, :] = lb3
            pos = (lb3 > 0.0).astype(jnp.float32).sum(axis=2)
            cnt_rows.append(pos * exf[ib * IB:(ib + 1) * IB][:, None]
                            * exf[None, :])
        cnt = jnp.concatenate(cnt_rows, axis=0)
        has_edges = jnp.any(cnt > 0.0)

        cf = child
        feats = [child]
        for it in range(ITERS):
            a = jnp.dot(cf, wne_ref[it, :H],
                        preferred_element_type=jnp.float32) + bne_ref[it][None, :]
            b = jnp.dot(cf, wne_ref[it, H:], preferred_element_type=jnp.float32)
            rows = []
            for ib in range(C // IB):
                m = jnp.maximum(a[ib * IB:(ib + 1) * IB][:, None, :]
                                + b[None, :, :], 0.0)
                w = cnt[ib * IB:(ib + 1) * IB][:, :, None]
                rows.append(jnp.sum(m * w, axis=1))
            agg = jnp.concatenate(rows, axis=0)
            cf = jnp.where(has_edges, agg, cf)
            feats.append(cf)

        cf3 = jnp.concatenate(feats, axis=1)
        h = jnp.maximum(jnp.dot(cf3, wch_ref[...],
                                preferred_element_type=jnp.float32)
                        + bch_ref[...], 0.0)
        sem_ref[...] = (jnp.dot(h, wsem_ref[...],
                                preferred_element_type=jnp.float32)
                        + bsem_ref[...])[None]
        out_ref[...] = jnp.maximum(jnp.dot(h, wch2_ref[...],
                                           preferred_element_type=jnp.float32)
                                   + bch2_ref[...], 0.0)[None]


def kernel(parent_feature, W_parent, b_parent, W_exists, b_exists, W_el, b_el,
           W_ee, b_ee, W_ne, b_ne, W_child, b_child, W_sem, b_sem,
           W_child2, b_child2):
    f32 = jnp.float32
    wee2 = W_ee[:, :, 0].T
    bee2 = b_ee[:, 0][None, :]
    child_flat = _sc_matvec(parent_feature.reshape(F), W_parent, b_parent)
    full = lambda s: pl.BlockSpec(s, lambda *_: (0,) * len(s))
    out, sem, exists_logits, elog = pl.pallas_call(
        _body,
        in_specs=[
            full((C, H)),
            full((H, 1)), full((1, 1)),
            full((2 * H, H)), full((1, H)),
            full((H, ET)), full((1, ET)),
            full((ITERS, 2 * H, H)), full((ITERS, H)),
            full((H * (ITERS + 1), H)), full((1, H)),
            full((H, NSEM)), full((1, NSEM)),
            full((H, F)), full((1, F)),
        ],
        out_specs=[
            full((1, C, F)), full((1, C, NSEM)),
            full((1, C, 1)), full((1, C, C, ET)),
        ],
        out_shape=[
            jax.ShapeDtypeStruct((1, C, F), f32),
            jax.ShapeDtypeStruct((1, C, NSEM), f32),
            jax.ShapeDtypeStruct((1, C, 1), f32),
            jax.ShapeDtypeStruct((1, C, C, ET), f32),
        ],
    )(child_flat.reshape(C, H),
      W_exists, b_exists[None, :],
      W_el, b_el[None, :],
      wee2, bee2,
      W_ne, b_ne,
      W_child, b_child[None, :],
      W_sem, b_sem[None, :],
      W_child2, b_child2[None, :])
    return out, sem, exists_logits, elog


# speedup vs baseline: 1.8421x; 1.8421x over previous
import jax
import jax.numpy as jnp
from jax.experimental import pallas as pl
from jax.experimental.pallas import tpu as pltpu

C = 128
H = 256
F = 256
ITERS = 2
ET = 4
NSEM = 57
NBAND = 8
NRING = 4
IB = 16


def _body(parent_ref, wp_hbm, bp_ref, wex_ref, bex_ref, wel_ref, bel_ref,
          wee_ref, bee_ref, wne_ref, bne_ref, wch_ref, bch_ref,
          wsem_ref, bsem_ref, wch2_ref, bch2_ref,
          out_ref, sem_ref, exists_ref, elog_ref,
          wp_buf, child_ref, sems):
    RB = F // NBAND
    for k in range(NRING):
        pltpu.make_async_copy(wp_hbm.at[pl.ds(k * RB, RB), :],
                              wp_buf.at[k], sems.at[k]).start()
    pf = bp_ref[...][None, :].astype(jnp.float32)
    for k in range(NBAND):
        pltpu.make_async_copy(wp_hbm.at[pl.ds(k * RB, RB), :],
                              wp_buf.at[k % NRING], sems.at[k % NRING]).wait()
        pf = pf + jnp.dot(parent_ref[0, pl.ds(k * RB, RB)][None, :],
                          wp_buf[k % NRING],
                          preferred_element_type=jnp.float32)
        if k + NRING < NBAND:
            pltpu.make_async_copy(
                wp_hbm.at[pl.ds((k + NRING) * RB, RB), :],
                wp_buf.at[k % NRING], sems.at[k % NRING]).start()
    child_ref[...] = jnp.maximum(pf, 0.0).reshape(C, H)

    if True:
        child = child_ref[...]

        exl = jnp.dot(child, wex_ref[...],
                      preferred_element_type=jnp.float32) + bex_ref[...]
        exists_ref[...] = exl[None]
        exf = (exl[:, 0] > 0.0).astype(jnp.float32)

        wel = wel_ref[...]
        ea = jnp.dot(child, wel[:H],
                     preferred_element_type=jnp.float32) + bel_ref[...]
        eb = jnp.dot(child, wel[H:], preferred_element_type=jnp.float32)
        cnt_rows = []
        for ib in range(C // IB):
            el = jnp.maximum(ea[ib * IB:(ib + 1) * IB][:, None, :]
                             + eb[None, :, :], 0.0)
            lb = jnp.dot(el.reshape(IB * C, H), wee_ref[...],
                         preferred_element_type=jnp.float32) + bee_ref[...]
            lb3 = lb.reshape(IB, C, ET)
            elog_ref[0, pl.ds(ib * IB, IB), :, :] = lb3
            pos = (lb3 > 0.0).astype(jnp.float32).sum(axis=2)
            cnt_rows.append(pos * exf[ib * IB:(ib + 1) * IB][:, None]
                            * exf[None, :])
        cnt = jnp.concatenate(cnt_rows, axis=0)
        has_edges = jnp.any(cnt > 0.0)

        cf = child
        feats = [child]
        for it in range(ITERS):
            a = jnp.dot(cf, wne_ref[it, :H],
                        preferred_element_type=jnp.float32) + bne_ref[it][None, :]
            b = jnp.dot(cf, wne_ref[it, H:], preferred_element_type=jnp.float32)
            rows = []
            for ib in range(C // IB):
                m = jnp.maximum(a[ib * IB:(ib + 1) * IB][:, None, :]
                                + b[None, :, :], 0.0)
                wblk = cnt[ib * IB:(ib + 1) * IB]
                rows.append(jax.lax.dot_general(
                    wblk, m, (((1,), (1,)), ((0,), (0,))),
                    preferred_element_type=jnp.float32))
            agg = jnp.concatenate(rows, axis=0)
            cf = jnp.where(has_edges, agg, cf)
            feats.append(cf)

        cf3 = jnp.concatenate(feats, axis=1)
        h = jnp.maximum(jnp.dot(cf3, wch_ref[...],
                                preferred_element_type=jnp.float32)
                        + bch_ref[...], 0.0)
        sem_ref[...] = (jnp.dot(h, wsem_ref[...],
                                preferred_element_type=jnp.float32)
                        + bsem_ref[...])[None]
        out_ref[...] = jnp.maximum(jnp.dot(h, wch2_ref[...],
                                           preferred_element_type=jnp.float32)
                                   + bch2_ref[...], 0.0)[None]


def kernel(parent_feature, W_parent, b_parent, W_exists, b_exists, W_el, b_el,
           W_ee, b_ee, W_ne, b_ne, W_child, b_child, W_sem, b_sem,
           W_child2, b_child2):
    f32 = jnp.float32
    wee2 = W_ee[:, :, 0].T
    bee2 = b_ee[:, 0][None, :]
    full = lambda s: pl.BlockSpec(s, lambda *_: (0,) * len(s))
    out, sem, exists_logits, elog = pl.pallas_call(
        _body,
        in_specs=[
            full((1, F)),
            pl.BlockSpec(memory_space=pl.ANY),
            full((C * H,)),
            full((H, 1)), full((1, 1)),
            full((2 * H, H)), full((1, H)),
            full((H, ET)), full((1, ET)),
            full((ITERS, 2 * H, H)), full((ITERS, H)),
            full((H * (ITERS + 1), H)), full((1, H)),
            full((H, NSEM)), full((1, NSEM)),
            full((H, F)), full((1, F)),
        ],
        out_specs=[
            full((1, C, F)), full((1, C, NSEM)),
            full((1, C, 1)), full((1, C, C, ET)),
        ],
        out_shape=[
            jax.ShapeDtypeStruct((1, C, F), f32),
            jax.ShapeDtypeStruct((1, C, NSEM), f32),
            jax.ShapeDtypeStruct((1, C, 1), f32),
            jax.ShapeDtypeStruct((1, C, C, ET), f32),
        ],
        scratch_shapes=[pltpu.VMEM((NRING, F // NBAND, C * H), f32),
                        pltpu.VMEM((C, H), f32),
                        pltpu.SemaphoreType.DMA((NRING,))],
    )(parent_feature, W_parent, b_parent,
      W_exists, b_exists[None, :],
      W_el, b_el[None, :],
      wee2, bee2,
      W_ne, b_ne,
      W_child, b_child[None, :],
      W_sem, b_sem[None, :],
      W_child2, b_child2[None, :])
    return out, sem, exists_logits, elog


# speedup vs baseline: 1.9376x; 1.0519x over previous
import jax
import jax.numpy as jnp
from jax.experimental import pallas as pl
from jax.experimental.pallas import tpu as pltpu

C = 128
H = 256
F = 256
ITERS = 2
ET = 4
NSEM = 57
NBAND = 8
NRING = 4
IB = 32


def _body(parent_ref, wp_hbm, bp_ref, wex_ref, bex_ref, wel_ref, bel_ref,
          wee_ref, bee_ref, wne_ref, bne_ref, wch_ref, bch_ref,
          wsem_ref, bsem_ref, wch2_ref, bch2_ref,
          out_ref, sem_ref, exists_ref, elog_ref,
          wp_buf, child_ref, sems):
    RB = F // NBAND
    for k in range(NRING):
        pltpu.make_async_copy(wp_hbm.at[pl.ds(k * RB, RB), :],
                              wp_buf.at[k], sems.at[k]).start()
    pf = bp_ref[...][None, :].astype(jnp.float32)
    for k in range(NBAND):
        pltpu.make_async_copy(wp_hbm.at[pl.ds(k * RB, RB), :],
                              wp_buf.at[k % NRING], sems.at[k % NRING]).wait()
        pf = pf + jnp.dot(parent_ref[0, pl.ds(k * RB, RB)][None, :],
                          wp_buf[k % NRING],
                          preferred_element_type=jnp.float32)
        if k + NRING < NBAND:
            pltpu.make_async_copy(
                wp_hbm.at[pl.ds((k + NRING) * RB, RB), :],
                wp_buf.at[k % NRING], sems.at[k % NRING]).start()
    child_ref[...] = jnp.maximum(pf, 0.0).reshape(C, H)

    if True:
        child = child_ref[...]

        exl = jnp.dot(child, wex_ref[...],
                      preferred_element_type=jnp.float32) + bex_ref[...]
        exists_ref[...] = exl[None]
        exc = (exl > 0.0).astype(jnp.float32)
        exr = jnp.transpose(exc, (1, 0))

        wel = wel_ref[...]
        ea = jnp.dot(child, wel[:H],
                     preferred_element_type=jnp.float32) + bel_ref[...]
        eb = jnp.dot(child, wel[H:], preferred_element_type=jnp.float32)
        cnt_rows = []
        for ib in range(C // IB):
            el = jnp.maximum(ea[ib * IB:(ib + 1) * IB][:, None, :]
                             + eb[None, :, :], 0.0)
            lb = jnp.dot(el.reshape(IB * C, H), wee_ref[...],
                         preferred_element_type=jnp.float32) + bee_ref[...]
            lb3 = lb.reshape(IB, C, ET)
            elog_ref[0, pl.ds(ib * IB, IB), :, :] = lb3
            posf = (lb3 > 0.0).astype(jnp.float32)
            pos = ((posf[:, :, 0] + posf[:, :, 1])
                   + (posf[:, :, 2] + posf[:, :, 3]))
            cnt_rows.append(pos * exc[ib * IB:(ib + 1) * IB] * exr)
        cnt = jnp.concatenate(cnt_rows, axis=0)
        has_edges = jnp.any(cnt > 0.0)

        cf = child
        feats = [child]
        for it in range(ITERS):
            a = jnp.dot(cf, wne_ref[it, :H],
                        preferred_element_type=jnp.float32) + bne_ref[it][None, :]
            b = jnp.dot(cf, wne_ref[it, H:], preferred_element_type=jnp.float32)
            rows = []
            for ib in range(C // IB):
                m = jnp.maximum(a[ib * IB:(ib + 1) * IB][:, None, :]
                                + b[None, :, :], 0.0)
                wblk = cnt[ib * IB:(ib + 1) * IB]
                rows.append(jax.lax.dot_general(
                    wblk, m, (((1,), (1,)), ((0,), (0,))),
                    preferred_element_type=jnp.float32))
            agg = jnp.concatenate(rows, axis=0)
            cf = jnp.where(has_edges, agg, cf)
            feats.append(cf)

        cf3 = jnp.concatenate(feats, axis=1)
        h = jnp.maximum(jnp.dot(cf3, wch_ref[...],
                                preferred_element_type=jnp.float32)
                        + bch_ref[...], 0.0)
        sem_ref[...] = (jnp.dot(h, wsem_ref[...],
                                preferred_element_type=jnp.float32)
                        + bsem_ref[...])[None]
        out_ref[...] = jnp.maximum(jnp.dot(h, wch2_ref[...],
                                           preferred_element_type=jnp.float32)
                                   + bch2_ref[...], 0.0)[None]


def kernel(parent_feature, W_parent, b_parent, W_exists, b_exists, W_el, b_el,
           W_ee, b_ee, W_ne, b_ne, W_child, b_child, W_sem, b_sem,
           W_child2, b_child2):
    f32 = jnp.float32
    wee2 = W_ee[:, :, 0].T
    bee2 = b_ee[:, 0][None, :]
    full = lambda s: pl.BlockSpec(s, lambda *_: (0,) * len(s))
    out, sem, exists_logits, elog = pl.pallas_call(
        _body,
        in_specs=[
            full((1, F)),
            pl.BlockSpec(memory_space=pl.ANY),
            full((C * H,)),
            full((H, 1)), full((1, 1)),
            full((2 * H, H)), full((1, H)),
            full((H, ET)), full((1, ET)),
            full((ITERS, 2 * H, H)), full((ITERS, H)),
            full((H * (ITERS + 1), H)), full((1, H)),
            full((H, NSEM)), full((1, NSEM)),
            full((H, F)), full((1, F)),
        ],
        out_specs=[
            full((1, C, F)), full((1, C, NSEM)),
            full((1, C, 1)), full((1, C, C, ET)),
        ],
        out_shape=[
            jax.ShapeDtypeStruct((1, C, F), f32),
            jax.ShapeDtypeStruct((1, C, NSEM), f32),
            jax.ShapeDtypeStruct((1, C, 1), f32),
            jax.ShapeDtypeStruct((1, C, C, ET), f32),
        ],
        scratch_shapes=[pltpu.VMEM((NRING, F // NBAND, C * H), f32),
                        pltpu.VMEM((C, H), f32),
                        pltpu.SemaphoreType.DMA((NRING,))],
    )(parent_feature, W_parent, b_parent,
      W_exists, b_exists[None, :],
      W_el, b_el[None, :],
      wee2, bee2,
      W_ne, b_ne,
      W_child, b_child[None, :],
      W_sem, b_sem[None, :],
      W_child2, b_child2[None, :])
    return out, sem, exists_logits, elog
